# baseline (speedup 1.0000x reference)
import jax
import jax.numpy as jnp
from jax.experimental import pallas as pl
from jax.experimental.pallas import tpu as pltpu
from jax.experimental.pallas import tpu_sc as plsc

BLOCK_ROWS = 2048
D = 1024


def _sc_bias(lvl, level_embeddings, band_embeddings):
    mesh = plsc.VectorSubcoreMesh(core_axis_name="c", subcore_axis_name="s")

    @pl.kernel(
        out_type=jax.ShapeDtypeStruct((1, D), jnp.float32),
        mesh=mesh,
        scratch_types=[
            pltpu.VMEM((1, D), jnp.float32),
            pltpu.VMEM((1, D), jnp.float32),
            pltpu.VMEM((1, 1), jnp.int32),
        ],
    )
    def bias_kernel(lvl_hbm, lev_hbm, band_hbm, o_hbm, a_v, b_v, i_v):
        c = jax.lax.axis_index("c")
        s = jax.lax.axis_index("s")

        @pl.when(jnp.logical_and(c == 0, s == 0))
        def _():
            pltpu.sync_copy(lvl_hbm, i_v)
            pltpu.sync_copy(lev_hbm.at[i_v.at[0]], a_v)
            pltpu.sync_copy(band_hbm.at[pl.ds(0, 1)], b_v)

            @pl.loop(0, D, step=16)
            def _(k):
                slc = (pl.ds(0, 1), pl.ds(k, 16))
                a_v.at[*slc][...] = a_v.at[*slc][...] + b_v.at[*slc][...]

            pltpu.sync_copy(a_v, o_hbm)

    return bias_kernel(lvl, level_embeddings, band_embeddings)


def _tc_add_kernel(x_ref, bias_ref, o_ref):
    o_ref[...] = x_ref[...] + bias_ref[...]


def kernel(x, level, level_embeddings, band_embeddings):
    b, s, d = x.shape
    rows = b * s
    x2 = x.reshape(rows, d)
    lvl = jnp.reshape(jnp.asarray(level, dtype=jnp.int32), (1, 1))
    bias = _sc_bias(lvl, level_embeddings, band_embeddings)
    out = pl.pallas_call(
        _tc_add_kernel,
        grid=(rows // BLOCK_ROWS,),
        in_specs=[
            pl.BlockSpec((BLOCK_ROWS, d), lambda i: (i, 0)),
            pl.BlockSpec((1, d), lambda i: (0, 0)),
        ],
        out_specs=pl.BlockSpec((BLOCK_ROWS, d), lambda i: (i, 0)),
        out_shape=jax.ShapeDtypeStruct((rows, d), x.dtype),
        compiler_params=pltpu.CompilerParams(
            dimension_semantics=("arbitrary",),
        ),
    )(x2, bias)
    return out.reshape(b, s, d)
